# baseline (speedup 1.0000x reference)
import dataclasses
import functools

import jax
import jax.numpy as jnp
from jax import lax
from jax.experimental import pallas as pl
from jax.experimental.pallas import tpu as pltpu
from jax.experimental.pallas import tpu_sc as plsc

N_NODES_K = 10000
DIM_K = 128
N_EDGES_K = 320000
NC = 2
NS = 16
NW = NC * NS
LANES = 16

_SC_PARAMS = pltpu.CompilerParams()
if "needs_layout_passes" in pltpu.CompilerParams.__dataclass_fields__:
    _SC_PARAMS = dataclasses.replace(_SC_PARAMS, needs_layout_passes=False)

EC = 80
EPW = N_EDGES_K // NW
N_PAD = 10240
N_PER_SUB = N_PAD // NS

PAIRS_PAD = 204800
PPW = PAIRS_PAD // NW
PC = 64


def _mm_body(x_ref, w_ref, o_ref):
    o_ref[...] = jnp.dot(x_ref[...], w_ref[...],
                         preferred_element_type=jnp.float32)


def _matmul_tc(x, w):
    return pl.pallas_call(
        _mm_body,
        out_shape=jax.ShapeDtypeStruct((x.shape[0], w.shape[1]), jnp.float32),
    )(x, w)


def _fused_relu_mm_body(p_ref, b_ref, w_ref, o_ref):
    h = jnp.maximum(p_ref[0] + p_ref[1] + b_ref[...], 0.0)
    o_ref[...] = jnp.dot(h, w_ref[...], preferred_element_type=jnp.float32)


def _fused_relu_mm_tc(p, b, w):
    return pl.pallas_call(
        _fused_relu_mm_body,
        out_shape=jax.ShapeDtypeStruct((p.shape[1], w.shape[1]), jnp.float32),
    )(p, b, w)


def _combine_body(p_ref, b_ref, o_ref):
    o_ref[...] = p_ref[0] + p_ref[1] + b_ref[...]


def _combine_tc(p, b):
    return pl.pallas_call(
        _combine_body,
        out_shape=jax.ShapeDtypeStruct((p.shape[1], p.shape[2]), jnp.float32),
    )(p, b)


def _spmm_sc(support, src, dst, vals, zeros):
    mesh = plsc.VectorSubcoreMesh(core_axis_name="c", subcore_axis_name="s")
    n = EPW // EC

    @functools.partial(
        pl.kernel,
        out_type=jax.ShapeDtypeStruct((NC, N_PAD, DIM_K), jnp.float32),
        mesh=mesh,
        compiler_params=_SC_PARAMS,
        scratch_types=[
            pltpu.VMEM((2, EC), jnp.int32),
            pltpu.VMEM((2, EC), jnp.int32),
            pltpu.VMEM((2, EC), jnp.float32),
            pltpu.VMEM((2, EC, DIM_K), jnp.float32),
            pltpu.VMEM_SHARED((N_PAD, DIM_K), jnp.float32),
            pltpu.SemaphoreType.DMA((2, 3)),
            pltpu.SemaphoreType.DMA((2,)),
        ],
    )
    def k(sup_hbm, src_hbm, dst_hbm, val_hbm, zero_hbm, out_hbm,
          sidx_v, didx_v, val_v, rows_v, acc, isem, gsem):
        cid = lax.axis_index("c")
        sid = lax.axis_index("s")
        wid = cid * NS + sid

        pltpu.sync_copy(zero_hbm, acc.at[pl.ds(sid * N_PER_SUB, N_PER_SUB)])
        plsc.subcore_barrier()

        base = wid * EPW

        def issue_idx(ci, b):
            off = base + ci * EC
            pltpu.async_copy(src_hbm.at[pl.ds(off, EC)], sidx_v.at[b],
                             isem.at[b, 0])
            pltpu.async_copy(dst_hbm.at[pl.ds(off, EC)], didx_v.at[b],
                             isem.at[b, 1])
            pltpu.async_copy(val_hbm.at[pl.ds(off, EC)], val_v.at[b],
                             isem.at[b, 2])

        def wait_idx(b):
            pltpu.make_async_copy(src_hbm.at[pl.ds(0, EC)], sidx_v.at[b],
                                  isem.at[b, 0]).wait()
            pltpu.make_async_copy(dst_hbm.at[pl.ds(0, EC)], didx_v.at[b],
                                  isem.at[b, 1]).wait()
            pltpu.make_async_copy(val_hbm.at[pl.ds(0, EC)], val_v.at[b],
                                  isem.at[b, 2]).wait()

        def issue_gather(b):
            pltpu.async_copy(sup_hbm.at[sidx_v.at[b]], rows_v.at[b],
                             gsem.at[b])

        def wait_gather(b):
            pltpu.make_async_copy(sup_hbm.at[sidx_v.at[b]], rows_v.at[b],
                                  gsem.at[b]).wait()

        def compute(b):
            @pl.loop(0, EC // LANES)
            def _grp(g):
                vg = val_v.at[b][pl.ds(g * LANES, LANES)]
                for l in range(LANES):
                    v = vg[l]
                    row = rows_v.at[b, g * LANES + l]
                    for j in range(DIM_K // LANES):
                        sl = pl.ds(j * LANES, LANES)
                        row[sl] = row[sl] * v

        issue_idx(0, 0)
        wait_idx(0)
        issue_gather(0)
        issue_idx(1, 1)

        m = 2 * ((n + 1) // 2)

        @pl.loop(0, m, step=2)
        def _pipe(ci):
            for kk in range(2):
                cur = ci + kk
                b, nb = kk, 1 - kk

                @pl.when(cur < n)
                def _():
                    wait_gather(b)

                @pl.when(cur + 1 < n)
                def _():
                    wait_idx(nb)
                    issue_gather(nb)

                @pl.when(cur < n)
                def _():
                    compute(b)
                    pltpu.sync_copy(rows_v.at[b], acc.at[didx_v.at[b]],
                                    add=True)

                @pl.when(cur + 2 < n)
                def _():
                    issue_idx(cur + 2, b)

        plsc.subcore_barrier()
        pltpu.sync_copy(acc.at[pl.ds(sid * N_PER_SUB, N_PER_SUB)],
                        out_hbm.at[cid, pl.ds(sid * N_PER_SUB, N_PER_SUB)])

    return k(support, src, dst, vals, zeros)


def _edge_dot_sc(feat, src, dst):
    mesh = plsc.VectorSubcoreMesh(core_axis_name="c", subcore_axis_name="s")
    n = PPW // PC

    @functools.partial(
        pl.kernel,
        out_type=jax.ShapeDtypeStruct((PAIRS_PAD,), jnp.float32),
        mesh=mesh,
        compiler_params=_SC_PARAMS,
        scratch_types=[
            pltpu.VMEM((2, PC), jnp.int32),
            pltpu.VMEM((2, PC), jnp.int32),
            pltpu.VMEM((2, PC, DIM_K), jnp.float32),
            pltpu.VMEM((2, PC, DIM_K), jnp.float32),
            pltpu.VMEM((2, PC), jnp.float32),
            pltpu.VMEM_SHARED((N_PAD, DIM_K), jnp.float32),
            pltpu.SemaphoreType.DMA((2, 2)),
            pltpu.SemaphoreType.DMA((2, 2)),
            pltpu.SemaphoreType.DMA((2,)),
        ],
    )
    def k(feat_hbm, src_hbm, dst_hbm, out_hbm,
          sidx_v, didx_v, srows_v, drows_v, logit_v, sfeat, isem, gsem, wsem):
        cid = lax.axis_index("c")
        sid = lax.axis_index("s")
        wid = cid * NS + sid
        base = wid * PPW

        pltpu.sync_copy(feat_hbm.at[pl.ds(sid * N_PER_SUB, N_PER_SUB)],
                        sfeat.at[pl.ds(sid * N_PER_SUB, N_PER_SUB)])
        plsc.subcore_barrier()

        def issue_idx(ci, b):
            off = base + ci * PC
            pltpu.async_copy(src_hbm.at[pl.ds(off, PC)], sidx_v.at[b],
                             isem.at[b, 0])
            pltpu.async_copy(dst_hbm.at[pl.ds(off, PC)], didx_v.at[b],
                             isem.at[b, 1])

        def wait_idx(b):
            pltpu.make_async_copy(src_hbm.at[pl.ds(0, PC)], sidx_v.at[b],
                                  isem.at[b, 0]).wait()
            pltpu.make_async_copy(dst_hbm.at[pl.ds(0, PC)], didx_v.at[b],
                                  isem.at[b, 1]).wait()

        def issue_gather(b):
            pltpu.async_copy(sfeat.at[sidx_v.at[b]], srows_v.at[b],
                             gsem.at[b, 0])
            pltpu.async_copy(sfeat.at[didx_v.at[b]], drows_v.at[b],
                             gsem.at[b, 1])

        def wait_gather(b):
            pltpu.make_async_copy(sfeat.at[sidx_v.at[b]], srows_v.at[b],
                                  gsem.at[b, 0]).wait()
            pltpu.make_async_copy(sfeat.at[didx_v.at[b]], drows_v.at[b],
                                  gsem.at[b, 1]).wait()

        def issue_write(ci, b):
            off = base + ci * PC
            pltpu.async_copy(logit_v.at[b], out_hbm.at[pl.ds(off, PC)],
                             wsem.at[b])

        def wait_write(b):
            pltpu.make_async_copy(logit_v.at[b], out_hbm.at[pl.ds(0, PC)],
                                  wsem.at[b]).wait()

        def compute(b):
            @pl.loop(0, PC // LANES)
            def _grp(g):
                lane = lax.iota(jnp.int32, LANES)
                parts = []
                for l in range(LANES):
                    srow = srows_v.at[b, g * LANES + l]
                    drow = drows_v.at[b, g * LANES + l]
                    acc = srow[pl.ds(0, LANES)] * drow[pl.ds(0, LANES)]
                    for j in range(1, DIM_K // LANES):
                        sl = pl.ds(j * LANES, LANES)
                        acc = acc + srow[sl] * drow[sl]
                    parts.append(jnp.where(lane == l, jnp.sum(acc), 0.0))
                while len(parts) > 1:
                    parts = [a + c for a, c in zip(parts[::2], parts[1::2])]
                probs = 1.0 / (1.0 + jnp.exp(-parts[0]))
                logit_v.at[b][pl.ds(g * LANES, LANES)] = probs

        issue_idx(0, 0)
        wait_idx(0)
        issue_gather(0)
        issue_idx(1, 1)

        m = 2 * ((n + 1) // 2)

        @pl.loop(0, m, step=2)
        def _pipe(ci):
            for kk in range(2):
                cur = ci + kk
                b, nb = kk, 1 - kk

                @pl.when(cur < n)
                def _():
                    wait_gather(b)

                @pl.when(cur + 1 < n)
                def _():
                    wait_idx(nb)
                    issue_gather(nb)

                @pl.when(cur >= 2)
                def _():
                    wait_write(b)

                @pl.when(cur < n)
                def _():
                    compute(b)
                    issue_write(cur, b)

                @pl.when(cur + 2 < n)
                def _():
                    issue_idx(cur + 2, b)

        wait_write(0)
        wait_write(1)

    return k(feat, src, dst)


def kernel(adj_index, adj_values, Features, pos_edge_index, neg_edge_index,
           W1, b1, W3, b3):
    src = adj_index[0]
    dst = adj_index[1]
    zeros = jnp.zeros((N_PER_SUB, DIM_K), dtype=jnp.float32)
    b1r = b1.reshape(1, DIM_K)
    b3r = b3.reshape(1, DIM_K)

    support1 = _matmul_tc(Features, W1)
    p = _spmm_sc(support1, src, dst, adj_values, zeros)
    support2 = _fused_relu_mm_tc(p, b1r, W3)
    q = _spmm_sc(support2, src, dst, adj_values, zeros)
    out_feature = _combine_tc(q, b3r)

    n_pairs = pos_edge_index.shape[0] + neg_edge_index.shape[0]
    pad = PAIRS_PAD - n_pairs
    pad_idx = jnp.arange(pad, dtype=jnp.int32) % N_NODES_K
    psrc = jnp.concatenate([pos_edge_index[:, 0], neg_edge_index[:, 0],
                            pad_idx])
    pdst = jnp.concatenate([pos_edge_index[:, 1], neg_edge_index[:, 1],
                            pad_idx])
    probs = _edge_dot_sc(out_feature, psrc, pdst)
    return probs[:n_pairs]


# speedup vs baseline: 1.0855x; 1.0855x over previous
import dataclasses
import functools

import jax
import jax.numpy as jnp
from jax import lax
from jax.experimental import pallas as pl
from jax.experimental.pallas import tpu as pltpu
from jax.experimental.pallas import tpu_sc as plsc

N_NODES_K = 10000
DIM_K = 128
N_EDGES_K = 320000
NC = 2
NS = 16
NW = NC * NS
LANES = 16

_SC_PARAMS = pltpu.CompilerParams()
if "needs_layout_passes" in pltpu.CompilerParams.__dataclass_fields__:
    _SC_PARAMS = dataclasses.replace(_SC_PARAMS, needs_layout_passes=False)

EC = 80
EPW = N_EDGES_K // NW
N_PAD = 10240
N_PER_SUB = N_PAD // NS

PAIRS_PAD = 204800
PPW = PAIRS_PAD // NW
PC = 128


def _mm_body(x_ref, w_ref, o_ref):
    o_ref[...] = jnp.dot(x_ref[...], w_ref[...],
                         preferred_element_type=jnp.float32)


def _matmul_tc(x, w):
    return pl.pallas_call(
        _mm_body,
        out_shape=jax.ShapeDtypeStruct((x.shape[0], w.shape[1]), jnp.float32),
    )(x, w)


def _fused_relu_mm_body(p_ref, b_ref, w_ref, o_ref):
    h = jnp.maximum(p_ref[0] + p_ref[1] + b_ref[...], 0.0)
    o_ref[...] = jnp.dot(h, w_ref[...], preferred_element_type=jnp.float32)


def _fused_relu_mm_tc(p, b, w):
    return pl.pallas_call(
        _fused_relu_mm_body,
        out_shape=jax.ShapeDtypeStruct((p.shape[1], w.shape[1]), jnp.float32),
    )(p, b, w)


def _combine_body(p_ref, b_ref, o_ref):
    o_ref[...] = p_ref[0] + p_ref[1] + b_ref[...]


def _combine_tc(p, b):
    return pl.pallas_call(
        _combine_body,
        out_shape=jax.ShapeDtypeStruct((p.shape[1], p.shape[2]), jnp.float32),
    )(p, b)


def _spmm_sc(support, src, dst, vals, zeros):
    mesh = plsc.VectorSubcoreMesh(core_axis_name="c", subcore_axis_name="s")
    n = EPW // EC

    @functools.partial(
        pl.kernel,
        out_type=jax.ShapeDtypeStruct((NC, N_PAD, DIM_K), jnp.float32),
        mesh=mesh,
        compiler_params=_SC_PARAMS,
        scratch_types=[
            pltpu.VMEM((2, EC), jnp.int32),
            pltpu.VMEM((2, EC), jnp.int32),
            pltpu.VMEM((2, EC), jnp.float32),
            pltpu.VMEM((2, EC, DIM_K), jnp.float32),
            pltpu.VMEM_SHARED((N_PAD, DIM_K), jnp.float32),
            pltpu.SemaphoreType.DMA((2, 3)),
            pltpu.SemaphoreType.DMA((2,)),
        ],
    )
    def k(sup_hbm, src_hbm, dst_hbm, val_hbm, zero_hbm, out_hbm,
          sidx_v, didx_v, val_v, rows_v, acc, isem, gsem):
        cid = lax.axis_index("c")
        sid = lax.axis_index("s")
        wid = cid * NS + sid

        pltpu.sync_copy(zero_hbm, acc.at[pl.ds(sid * N_PER_SUB, N_PER_SUB)])
        plsc.subcore_barrier()

        base = wid * EPW

        def issue_idx(ci, b):
            off = base + ci * EC
            pltpu.async_copy(src_hbm.at[pl.ds(off, EC)], sidx_v.at[b],
                             isem.at[b, 0])
            pltpu.async_copy(dst_hbm.at[pl.ds(off, EC)], didx_v.at[b],
                             isem.at[b, 1])
            pltpu.async_copy(val_hbm.at[pl.ds(off, EC)], val_v.at[b],
                             isem.at[b, 2])

        def wait_idx(b):
            pltpu.make_async_copy(src_hbm.at[pl.ds(0, EC)], sidx_v.at[b],
                                  isem.at[b, 0]).wait()
            pltpu.make_async_copy(dst_hbm.at[pl.ds(0, EC)], didx_v.at[b],
                                  isem.at[b, 1]).wait()
            pltpu.make_async_copy(val_hbm.at[pl.ds(0, EC)], val_v.at[b],
                                  isem.at[b, 2]).wait()

        def issue_gather(b):
            pltpu.async_copy(sup_hbm.at[sidx_v.at[b]], rows_v.at[b],
                             gsem.at[b])

        def wait_gather(b):
            pltpu.make_async_copy(sup_hbm.at[sidx_v.at[b]], rows_v.at[b],
                                  gsem.at[b]).wait()

        def compute(b):
            @pl.loop(0, EC // LANES)
            def _grp(g):
                vg = val_v.at[b][pl.ds(g * LANES, LANES)]
                for l in range(LANES):
                    v = vg[l]
                    row = rows_v.at[b, g * LANES + l]
                    for j in range(DIM_K // LANES):
                        sl = pl.ds(j * LANES, LANES)
                        row[sl] = row[sl] * v

        issue_idx(0, 0)
        wait_idx(0)
        issue_gather(0)
        issue_idx(1, 1)

        m = 2 * ((n + 1) // 2)

        @pl.loop(0, m, step=2)
        def _pipe(ci):
            for kk in range(2):
                cur = ci + kk
                b, nb = kk, 1 - kk

                @pl.when(cur < n)
                def _():
                    wait_gather(b)

                @pl.when(cur + 1 < n)
                def _():
                    wait_idx(nb)
                    issue_gather(nb)

                @pl.when(cur < n)
                def _():
                    compute(b)
                    pltpu.sync_copy(rows_v.at[b], acc.at[didx_v.at[b]],
                                    add=True)

                @pl.when(cur + 2 < n)
                def _():
                    issue_idx(cur + 2, b)

        plsc.subcore_barrier()
        pltpu.sync_copy(acc.at[pl.ds(sid * N_PER_SUB, N_PER_SUB)],
                        out_hbm.at[cid, pl.ds(sid * N_PER_SUB, N_PER_SUB)])

    return k(support, src, dst, vals, zeros)


def _edge_dot_sc(feat, src, dst):
    mesh = plsc.VectorSubcoreMesh(core_axis_name="c", subcore_axis_name="s")
    n = PPW // PC

    @functools.partial(
        pl.kernel,
        out_type=jax.ShapeDtypeStruct((PAIRS_PAD,), jnp.float32),
        mesh=mesh,
        compiler_params=_SC_PARAMS,
        scratch_types=[
            pltpu.VMEM((2, PC), jnp.int32),
            pltpu.VMEM((2, PC), jnp.int32),
            pltpu.VMEM((2, PC, DIM_K), jnp.float32),
            pltpu.VMEM((2, PC, DIM_K), jnp.float32),
            pltpu.VMEM((2, PC), jnp.float32),
            pltpu.SemaphoreType.DMA((2, 2)),
            pltpu.SemaphoreType.DMA((2, 2)),
            pltpu.SemaphoreType.DMA((2,)),
        ],
    )
    def k(feat_hbm, src_hbm, dst_hbm, out_hbm,
          sidx_v, didx_v, srows_v, drows_v, logit_v, isem, gsem, wsem):
        cid = lax.axis_index("c")
        sid = lax.axis_index("s")
        wid = cid * NS + sid
        base = wid * PPW

        def issue_idx(ci, b):
            off = base + ci * PC
            pltpu.async_copy(src_hbm.at[pl.ds(off, PC)], sidx_v.at[b],
                             isem.at[b, 0])
            pltpu.async_copy(dst_hbm.at[pl.ds(off, PC)], didx_v.at[b],
                             isem.at[b, 1])

        def wait_idx(b):
            pltpu.make_async_copy(src_hbm.at[pl.ds(0, PC)], sidx_v.at[b],
                                  isem.at[b, 0]).wait()
            pltpu.make_async_copy(dst_hbm.at[pl.ds(0, PC)], didx_v.at[b],
                                  isem.at[b, 1]).wait()

        def issue_gather(b):
            pltpu.async_copy(feat_hbm.at[sidx_v.at[b]], srows_v.at[b],
                             gsem.at[b, 0])
            pltpu.async_copy(feat_hbm.at[didx_v.at[b]], drows_v.at[b],
                             gsem.at[b, 1])

        def wait_gather(b):
            pltpu.make_async_copy(feat_hbm.at[sidx_v.at[b]], srows_v.at[b],
                                  gsem.at[b, 0]).wait()
            pltpu.make_async_copy(feat_hbm.at[didx_v.at[b]], drows_v.at[b],
                                  gsem.at[b, 1]).wait()

        def issue_write(ci, b):
            off = base + ci * PC
            pltpu.async_copy(logit_v.at[b], out_hbm.at[pl.ds(off, PC)],
                             wsem.at[b])

        def wait_write(b):
            pltpu.make_async_copy(logit_v.at[b], out_hbm.at[pl.ds(0, PC)],
                                  wsem.at[b]).wait()

        def compute(b):
            @pl.loop(0, PC // LANES)
            def _grp(g):
                lane = lax.iota(jnp.int32, LANES)
                parts = []
                for l in range(LANES):
                    srow = srows_v.at[b, g * LANES + l]
                    drow = drows_v.at[b, g * LANES + l]
                    acc = srow[pl.ds(0, LANES)] * drow[pl.ds(0, LANES)]
                    for j in range(1, DIM_K // LANES):
                        sl = pl.ds(j * LANES, LANES)
                        acc = acc + srow[sl] * drow[sl]
                    parts.append(jnp.where(lane == l, jnp.sum(acc), 0.0))
                while len(parts) > 1:
                    parts = [a + c for a, c in zip(parts[::2], parts[1::2])]
                probs = 1.0 / (1.0 + jnp.exp(-parts[0]))
                logit_v.at[b][pl.ds(g * LANES, LANES)] = probs

        issue_idx(0, 0)
        wait_idx(0)
        issue_gather(0)
        issue_idx(1, 1)

        m = 2 * ((n + 1) // 2)

        @pl.loop(0, m, step=2)
        def _pipe(ci):
            for kk in range(2):
                cur = ci + kk
                b, nb = kk, 1 - kk

                @pl.when(cur < n)
                def _():
                    wait_gather(b)

                @pl.when(cur + 1 < n)
                def _():
                    wait_idx(nb)
                    issue_gather(nb)

                @pl.when(cur >= 2)
                def _():
                    wait_write(b)

                @pl.when(cur < n)
                def _():
                    compute(b)
                    issue_write(cur, b)

                @pl.when(cur + 2 < n)
                def _():
                    issue_idx(cur + 2, b)

        wait_write(0)
        wait_write(1)

    return k(feat, src, dst)


def kernel(adj_index, adj_values, Features, pos_edge_index, neg_edge_index,
           W1, b1, W3, b3):
    src = adj_index[0]
    dst = adj_index[1]
    zeros = jnp.zeros((N_PER_SUB, DIM_K), dtype=jnp.float32)
    b1r = b1.reshape(1, DIM_K)
    b3r = b3.reshape(1, DIM_K)

    support1 = _matmul_tc(Features, W1)
    p = _spmm_sc(support1, src, dst, adj_values, zeros)
    support2 = _fused_relu_mm_tc(p, b1r, W3)
    q = _spmm_sc(support2, src, dst, adj_values, zeros)
    out_feature = _combine_tc(q, b3r)

    n_pairs = pos_edge_index.shape[0] + neg_edge_index.shape[0]
    pad = PAIRS_PAD - n_pairs
    pad_idx = jnp.arange(pad, dtype=jnp.int32) % N_NODES_K
    psrc = jnp.concatenate([pos_edge_index[:, 0], neg_edge_index[:, 0],
                            pad_idx])
    pdst = jnp.concatenate([pos_edge_index[:, 1], neg_edge_index[:, 1],
                            pad_idx])
    probs = _edge_dot_sc(out_feature, psrc, pdst)
    return probs[:n_pairs]


# speedup vs baseline: 1.1472x; 1.0568x over previous
import dataclasses
import functools

import jax
import jax.numpy as jnp
from jax import lax
from jax.experimental import pallas as pl
from jax.experimental.pallas import tpu as pltpu
from jax.experimental.pallas import tpu_sc as plsc

N_NODES_K = 10000
DIM_K = 128
N_EDGES_K = 320000
NC = 2
NS = 16
NW = NC * NS
LANES = 16

_SC_PARAMS = pltpu.CompilerParams()
if "needs_layout_passes" in pltpu.CompilerParams.__dataclass_fields__:
    _SC_PARAMS = dataclasses.replace(_SC_PARAMS, needs_layout_passes=False)

EC = 128
N_CHUNKS = N_EDGES_K // EC
N_PAD = 10240
N_PER_SUB = N_PAD // NS

PAIRS_PAD = 204800
PPW = PAIRS_PAD // NW
PC = 128


def _mm_body(x_ref, w_ref, o_ref):
    o_ref[...] = jnp.dot(x_ref[...], w_ref[...],
                         preferred_element_type=jnp.float32)


def _matmul_tc(x, w):
    return pl.pallas_call(
        _mm_body,
        out_shape=jax.ShapeDtypeStruct((x.shape[0], w.shape[1]), jnp.float32),
    )(x, w)


def _fused_relu_mm_body(p_ref, b_ref, w_ref, o_ref):
    h = jnp.maximum(p_ref[0] + p_ref[1] + b_ref[...], 0.0)
    o_ref[...] = jnp.dot(h, w_ref[...], preferred_element_type=jnp.float32)


def _fused_relu_mm_tc(p, b, w):
    return pl.pallas_call(
        _fused_relu_mm_body,
        out_shape=jax.ShapeDtypeStruct((p.shape[1], w.shape[1]), jnp.float32),
    )(p, b, w)


def _combine_body(p_ref, b_ref, o_ref):
    o_ref[...] = p_ref[0] + p_ref[1] + b_ref[...]


def _combine_tc(p, b):
    return pl.pallas_call(
        _combine_body,
        out_shape=jax.ShapeDtypeStruct((p.shape[1], p.shape[2]), jnp.float32),
    )(p, b)


def _spmm_sc(support, src, dst, vals, zeros):
    mesh = plsc.VectorSubcoreMesh(core_axis_name="c", subcore_axis_name="s")
    m_static = 2 * ((N_CHUNKS // NW + 2) // 2)

    @functools.partial(
        pl.kernel,
        out_type=jax.ShapeDtypeStruct((NC, N_PAD, DIM_K), jnp.float32),
        mesh=mesh,
        compiler_params=_SC_PARAMS,
        scratch_types=[
            pltpu.VMEM((2, EC), jnp.int32),
            pltpu.VMEM((2, EC), jnp.int32),
            pltpu.VMEM((2, EC), jnp.float32),
            pltpu.VMEM((2, EC, DIM_K), jnp.float32),
            pltpu.VMEM_SHARED((N_PAD, DIM_K), jnp.float32),
            pltpu.SemaphoreType.DMA((2, 3)),
            pltpu.SemaphoreType.DMA((2,)),
        ],
    )
    def k(sup_hbm, src_hbm, dst_hbm, val_hbm, zero_hbm, out_hbm,
          sidx_v, didx_v, val_v, rows_v, acc, isem, gsem):
        cid = lax.axis_index("c")
        sid = lax.axis_index("s")
        wid = cid * NS + sid

        pltpu.sync_copy(zero_hbm, acc.at[pl.ds(sid * N_PER_SUB, N_PER_SUB)])
        plsc.subcore_barrier()

        n = (N_CHUNKS - wid + NW - 1) // NW

        def issue_idx(ci, b):
            off = (wid + ci * NW) * EC
            pltpu.async_copy(src_hbm.at[pl.ds(off, EC)], sidx_v.at[b],
                             isem.at[b, 0])
            pltpu.async_copy(dst_hbm.at[pl.ds(off, EC)], didx_v.at[b],
                             isem.at[b, 1])
            pltpu.async_copy(val_hbm.at[pl.ds(off, EC)], val_v.at[b],
                             isem.at[b, 2])

        def wait_idx(b):
            pltpu.make_async_copy(src_hbm.at[pl.ds(0, EC)], sidx_v.at[b],
                                  isem.at[b, 0]).wait()
            pltpu.make_async_copy(dst_hbm.at[pl.ds(0, EC)], didx_v.at[b],
                                  isem.at[b, 1]).wait()
            pltpu.make_async_copy(val_hbm.at[pl.ds(0, EC)], val_v.at[b],
                                  isem.at[b, 2]).wait()

        def issue_gather(b):
            pltpu.async_copy(sup_hbm.at[sidx_v.at[b]], rows_v.at[b],
                             gsem.at[b])

        def wait_gather(b):
            pltpu.make_async_copy(sup_hbm.at[sidx_v.at[b]], rows_v.at[b],
                                  gsem.at[b]).wait()

        def compute(b):
            @pl.loop(0, EC // LANES)
            def _grp(g):
                vg = val_v.at[b][pl.ds(g * LANES, LANES)]
                for l in range(LANES):
                    v = vg[l]
                    row = rows_v.at[b, g * LANES + l]
                    for j in range(DIM_K // LANES):
                        sl = pl.ds(j * LANES, LANES)
                        row[sl] = row[sl] * v

        issue_idx(0, 0)
        wait_idx(0)
        issue_gather(0)
        issue_idx(1, 1)

        @pl.loop(0, m_static, step=2)
        def _pipe(ci):
            for kk in range(2):
                cur = ci + kk
                b, nb = kk, 1 - kk

                @pl.when(cur < n)
                def _():
                    wait_gather(b)

                @pl.when(cur + 1 < n)
                def _():
                    wait_idx(nb)
                    issue_gather(nb)

                @pl.when(cur < n)
                def _():
                    compute(b)
                    pltpu.sync_copy(rows_v.at[b], acc.at[didx_v.at[b]],
                                    add=True)

                @pl.when(cur + 2 < n)
                def _():
                    issue_idx(cur + 2, b)

        plsc.subcore_barrier()
        pltpu.sync_copy(acc.at[pl.ds(sid * N_PER_SUB, N_PER_SUB)],
                        out_hbm.at[cid, pl.ds(sid * N_PER_SUB, N_PER_SUB)])

    return k(support, src, dst, vals, zeros)


def _edge_dot_sc(feat, src, dst):
    mesh = plsc.VectorSubcoreMesh(core_axis_name="c", subcore_axis_name="s")
    n = PPW // PC

    @functools.partial(
        pl.kernel,
        out_type=jax.ShapeDtypeStruct((PAIRS_PAD,), jnp.float32),
        mesh=mesh,
        compiler_params=_SC_PARAMS,
        scratch_types=[
            pltpu.VMEM((2, PC), jnp.int32),
            pltpu.VMEM((2, PC), jnp.int32),
            pltpu.VMEM((2, PC, DIM_K), jnp.float32),
            pltpu.VMEM((2, PC, DIM_K), jnp.float32),
            pltpu.VMEM((2, PC), jnp.float32),
            pltpu.SemaphoreType.DMA((2, 2)),
            pltpu.SemaphoreType.DMA((2, 2)),
            pltpu.SemaphoreType.DMA((2,)),
        ],
    )
    def k(feat_hbm, src_hbm, dst_hbm, out_hbm,
          sidx_v, didx_v, srows_v, drows_v, logit_v, isem, gsem, wsem):
        cid = lax.axis_index("c")
        sid = lax.axis_index("s")
        wid = cid * NS + sid
        base = wid * PPW

        def issue_idx(ci, b):
            off = base + ci * PC
            pltpu.async_copy(src_hbm.at[pl.ds(off, PC)], sidx_v.at[b],
                             isem.at[b, 0])
            pltpu.async_copy(dst_hbm.at[pl.ds(off, PC)], didx_v.at[b],
                             isem.at[b, 1])

        def wait_idx(b):
            pltpu.make_async_copy(src_hbm.at[pl.ds(0, PC)], sidx_v.at[b],
                                  isem.at[b, 0]).wait()
            pltpu.make_async_copy(dst_hbm.at[pl.ds(0, PC)], didx_v.at[b],
                                  isem.at[b, 1]).wait()

        def issue_gather(b):
            pltpu.async_copy(feat_hbm.at[sidx_v.at[b]], srows_v.at[b],
                             gsem.at[b, 0])
            pltpu.async_copy(feat_hbm.at[didx_v.at[b]], drows_v.at[b],
                             gsem.at[b, 1])

        def wait_gather(b):
            pltpu.make_async_copy(feat_hbm.at[sidx_v.at[b]], srows_v.at[b],
                                  gsem.at[b, 0]).wait()
            pltpu.make_async_copy(feat_hbm.at[didx_v.at[b]], drows_v.at[b],
                                  gsem.at[b, 1]).wait()

        def issue_write(ci, b):
            off = base + ci * PC
            pltpu.async_copy(logit_v.at[b], out_hbm.at[pl.ds(off, PC)],
                             wsem.at[b])

        def wait_write(b):
            pltpu.make_async_copy(logit_v.at[b], out_hbm.at[pl.ds(0, PC)],
                                  wsem.at[b]).wait()

        def compute(b):
            @pl.loop(0, PC // LANES)
            def _grp(g):
                lane = lax.iota(jnp.int32, LANES)
                parts = []
                for l in range(LANES):
                    srow = srows_v.at[b, g * LANES + l]
                    drow = drows_v.at[b, g * LANES + l]
                    acc = srow[pl.ds(0, LANES)] * drow[pl.ds(0, LANES)]
                    for j in range(1, DIM_K // LANES):
                        sl = pl.ds(j * LANES, LANES)
                        acc = acc + srow[sl] * drow[sl]
                    parts.append(jnp.where(lane == l, jnp.sum(acc), 0.0))
                while len(parts) > 1:
                    parts = [a + c for a, c in zip(parts[::2], parts[1::2])]
                probs = 1.0 / (1.0 + jnp.exp(-parts[0]))
                logit_v.at[b][pl.ds(g * LANES, LANES)] = probs

        issue_idx(0, 0)
        wait_idx(0)
        issue_gather(0)
        issue_idx(1, 1)

        m = 2 * ((n + 1) // 2)

        @pl.loop(0, m, step=2)
        def _pipe(ci):
            for kk in range(2):
                cur = ci + kk
                b, nb = kk, 1 - kk

                @pl.when(cur < n)
                def _():
                    wait_gather(b)

                @pl.when(cur + 1 < n)
                def _():
                    wait_idx(nb)
                    issue_gather(nb)

                @pl.when(cur >= 2)
                def _():
                    wait_write(b)

                @pl.when(cur < n)
                def _():
                    compute(b)
                    issue_write(cur, b)

                @pl.when(cur + 2 < n)
                def _():
                    issue_idx(cur + 2, b)

        wait_write(0)
        wait_write(1)

    return k(feat, src, dst)


def kernel(adj_index, adj_values, Features, pos_edge_index, neg_edge_index,
           W1, b1, W3, b3):
    src = adj_index[0]
    dst = adj_index[1]
    zeros = jnp.zeros((N_PER_SUB, DIM_K), dtype=jnp.float32)
    b1r = b1.reshape(1, DIM_K)
    b3r = b3.reshape(1, DIM_K)

    support1 = _matmul_tc(Features, W1)
    p = _spmm_sc(support1, src, dst, adj_values, zeros)
    support2 = _fused_relu_mm_tc(p, b1r, W3)
    q = _spmm_sc(support2, src, dst, adj_values, zeros)
    out_feature = _combine_tc(q, b3r)

    n_pairs = pos_edge_index.shape[0] + neg_edge_index.shape[0]
    pad = PAIRS_PAD - n_pairs
    pad_idx = jnp.arange(pad, dtype=jnp.int32) % N_NODES_K
    psrc = jnp.concatenate([pos_edge_index[:, 0], neg_edge_index[:, 0],
                            pad_idx])
    pdst = jnp.concatenate([pos_edge_index[:, 1], neg_edge_index[:, 1],
                            pad_idx])
    probs = _edge_dot_sc(out_feature, psrc, pdst)
    return probs[:n_pairs]
